# Initial kernel scaffold; baseline (speedup 1.0000x reference)
import jax
import jax.numpy as jnp
from jax.experimental import pallas as pl


def kernel(x, edge_index, first_index, second_index, W, M, U, V, fc1_w, fc1_b, fc2_w, fc2_b):
    raise NotImplementedError("write your pallas kernel here")


# speedup vs baseline: 254.5829x; 254.5829x over previous
import functools

import jax
import jax.numpy as jnp
from jax import lax
from jax.experimental import pallas as pl
from jax.experimental.pallas import tpu as pltpu
from jax.experimental.pallas import tpu_sc as plsc

N = 100000
E = 3200000
L = 16
NC = 2
NS = 16
NW = NC * NS
EPW = E // NW
GROUPS = EPW // L
BIG = 2**30


def _sc_partial_sums(dst, src, x0, x1, x2, i1_arr, i2_arr):
    mesh = plsc.VectorSubcoreMesh(core_axis_name="c", subcore_axis_name="s")

    @functools.partial(
        pl.kernel,
        out_type=jax.ShapeDtypeStruct((NW, 8, L), jnp.float32),
        mesh=mesh,
        scratch_types=[
            pltpu.VMEM((EPW,), jnp.int32),
            pltpu.VMEM((L,), jnp.int32),
            pltpu.VMEM((L,), jnp.int32),
            pltpu.VMEM((L,), jnp.int32),
            pltpu.VMEM((L,), jnp.int32),
            pltpu.VMEM((3, L), jnp.float32),
            pltpu.VMEM((8, L), jnp.float32),
            pltpu.SemaphoreType.DMA,
        ],
    )
    def kern(dst_hbm, src_hbm, x0_hbm, x1_hbm, x2_hbm, i1_hbm, i2_hbm,
             out_hbm, dstv, i1v_ref, i2v_ref, sbuf, dbuf, gbuf, outv, gsem):
        wid = lax.axis_index("s") * NC + lax.axis_index("c")
        base = wid * EPW
        pltpu.sync_copy(i1_hbm, i1v_ref)
        pltpu.sync_copy(i2_hbm, i2v_ref)
        pltpu.sync_copy(dst_hbm.at[pl.ds(base, EPW)], dstv)
        i1v = i1v_ref[...]
        i2v = i2v_ref[...]
        zf = jnp.zeros((L,), jnp.float32)
        zi = jnp.zeros((L,), jnp.int32)
        onei = jnp.full((L,), 1, jnp.int32)
        bigv = jnp.full((L,), jnp.int32(BIG), jnp.int32)
        lane = lax.iota(jnp.int32, L)

        def scan(w):
            wv = jnp.full((L,), w, jnp.int32)

            def body(g, c):
                gidx, cnt = c
                d = dstv[pl.ds(g * L, L)]
                t = jnp.minimum(d ^ i1v, d ^ i2v)
                hitv = t == 0
                cap = jnp.where(cnt == wv, jnp.full((L,), g, jnp.int32), gidx)
                gidx = jnp.where(hitv, cap, gidx)
                cnt = cnt + jnp.where(hitv, onei, zi)
                return gidx, cnt

            return lax.fori_loop(0, GROUPS, body,
                                 (jnp.full((L,), -1, jnp.int32), zi),
                                 unroll=8)

        def vmin16(v):
            r = v[0]
            for i in range(1, L):
                r = jnp.minimum(r, v[i])
            return r

        def vmax16(v):
            r = v[0]
            for i in range(1, L):
                r = jnp.maximum(r, v[i])
            return r

        def wave(gidx, m):
            eidx = jnp.where(m, gidx * L + lane, zi)
            cs = pltpu.async_copy(
                src_hbm.at[jnp.where(m, eidx + base, zi)], sbuf, gsem)
            cd = pltpu.async_copy(
                dst_hbm.at[jnp.where(m, eidx + base, zi)], dbuf, gsem)
            cs.wait()
            cd.wait()
            dval = jnp.where(m, dbuf[...], jnp.full((L,), -1, jnp.int32))
            xi = jnp.where(m, sbuf[...], zi)
            c0 = pltpu.async_copy(x0_hbm.at[xi], gbuf.at[0], gsem)
            c1 = pltpu.async_copy(x1_hbm.at[xi], gbuf.at[1], gsem)
            c2 = pltpu.async_copy(x2_hbm.at[xi], gbuf.at[2], gsem)
            c0.wait()
            c1.wait()
            c2.wait()
            m1 = dval == i1v
            m2 = dval == i2v
            for d in range(3):
                outv[d, :] = outv[d, :] + jnp.where(m1, gbuf[d, :], zf)
                outv[3 + d, :] = outv[3 + d, :] + jnp.where(m2, gbuf[d, :], zf)

        for r in range(8):
            outv[r, :] = zf
        gidx, cnt = scan(0)
        mx = vmax16(cnt)

        @pl.when(mx > 0)
        def _():
            wave(gidx, cnt > 0)

        @pl.loop(1, mx)
        def _(w):
            gidx2, cnt2 = scan(w)
            wave(gidx2, cnt2 > w)
        pltpu.sync_copy(outv, out_hbm.at[wid])

    return kern(dst, src, x0, x1, x2, i1_arr, i2_arr)


def kernel(x, edge_index, first_index, second_index,
           W, M, U, V, fc1_w, fc1_b, fc2_w, fc2_b):
    src = edge_index[0]
    dst = edge_index[1]
    i1 = jnp.asarray(first_index, jnp.int32)
    i2 = jnp.asarray(second_index, jnp.int32)
    x0 = x[:, 0]
    x1 = x[:, 1]
    x2 = x[:, 2]
    i1_arr = jnp.full((L,), i1, jnp.int32)
    i2_arr = jnp.full((L,), i2, jnp.int32)

    parts = _sc_partial_sums(dst, src, x0, x1, x2, i1_arr, i2_arr)
    s1 = jnp.sum(parts[:, 0:3, :], axis=(0, 2))
    s2 = jnp.sum(parts[:, 3:6, :], axis=(0, 2))

    xr1 = x[i1]
    xr2 = x[i2]
    h1 = jax.nn.relu(xr1 @ U + s1 @ V)
    h2 = jax.nn.relu(xr2 @ U + s2 @ V)
    g1 = jax.nn.softmax(h1[None, :], axis=-1)
    g2 = jax.nn.softmax(h2[None, :], axis=-1)
    third = jnp.concatenate([g1 * g2, g1 + g2], axis=1)
    v = jax.nn.relu(third @ fc1_w.T + fc1_b)
    out = jax.nn.relu(v @ fc2_w.T + fc2_b)
    return out
